# baseline (speedup 1.0000x reference)
import jax
import jax.numpy as jnp
from jax import lax
from jax.experimental import pallas as pl
from jax.experimental.pallas import tpu as pltpu
from jax.experimental.pallas import tpu_sc as plsc

VOCAB_N = 100000
EMB_N = 32
BATCH_N = 1024
HIST_N = 200

NC = 2
NS = 16
NW = NC * NS
NSLICE = 4
SLICE_V = VOCAB_N // NSLICE
AROWS = BATCH_N // NW
RGS = AROWS // 8
BROWS = BATCH_N // NW
W = 6144
NK = 4
TAILC0 = NK * W
TAILW = SLICE_V - TAILC0
HPAD = 104
CH_OFFS = (0, 16, 32, 48, 64, 80, 88)


def _onehot_body(tok_hbm, out_hbm, tok_v, buf0, buf1, buft,
                 sem0, sem1, semt):
    wid = lax.axis_index("s") * NC + lax.axis_index("c")
    base_row = wid * AROWS
    base_rg = wid * RGS

    pltpu.sync_copy(tok_hbm.at[pl.ds(base_row, AROWS)], tok_v)

    zero16 = jnp.zeros((16,), jnp.float32)
    one16 = jnp.ones((16,), jnp.float32)

    def z_main(i, carry):
        for rr in range(8):
            buf0[rr, pl.ds(i * 16, 16)] = zero16
            buf1[rr, pl.ds(i * 16, 16)] = zero16
        return carry
    lax.fori_loop(0, W // 16, z_main, 0)

    tail_offs = tuple(range(0, TAILW - 16, 16)) + (TAILW - 16,)

    def z_tail(rr, carry):
        for off in tail_offs:
            buft[rr, pl.ds(off, 16)] = zero16
        return carry
    lax.fori_loop(0, 8, z_tail, 0)

    def scan(buf, rg, c0, cw, val16):
        def rbody(rr, carry):
            row = rg * 8 + rr
            ir = jnp.full((16,), rr, jnp.int32)
            for hh in range(2):
                for off in CH_OFFS:
                    t = tok_v[row, hh, pl.ds(off, 16)]
                    m = (t >= c0) & (t < c0 + cw)
                    ic = jnp.where(m, t - c0, 0)
                    plsc.store_scatter(buf, [ir, ic], val16, mask=m)
            return carry
        lax.fori_loop(0, 8, rbody, 0)

    bufs = (buf0, buf1)
    sems = (sem0, sem1)

    def rg_body(rg, carry):
        rgg = base_rg + rg

        def kp_body(j, c):
            for u in range(2):
                k = 2 * j + u
                c0 = k * W
                buf, sem = bufs[u], sems[u]

                @pl.when((k >= 2) | (rg > 0))
                def _reuse(buf=buf, sem=sem, k=k, c0=c0):
                    k2 = jnp.where(k >= 2, k - 2, NK - 2 + k)
                    rg2 = jnp.where(k >= 2, rg, rg - 1)
                    pltpu.make_async_copy(
                        buf, out_hbm.at[pl.ds(0, 8), pl.ds(0, W)], sem).wait()
                    scan(buf, rg2, k2 * W, W, zero16)

                scan(buf, rg, c0, W, one16)
                pltpu.async_copy(
                    buf, out_hbm.at[pl.ds(rgg * 8, 8), pl.ds(c0, W)], sem)
            return c
        lax.fori_loop(0, NK // 2, kp_body, 0)

        @pl.when(rg > 0)
        def _tail_reuse():
            pltpu.make_async_copy(
                buft, out_hbm.at[pl.ds(0, 8), pl.ds(TAILC0, TAILW)],
                semt).wait()
            scan(buft, rg - 1, TAILC0, TAILW, zero16)
        scan(buft, rg, TAILC0, TAILW, one16)
        pltpu.async_copy(
            buft, out_hbm.at[pl.ds(rgg * 8, 8), pl.ds(TAILC0, TAILW)], semt)
        return carry
    lax.fori_loop(0, RGS, rg_body, 0)

    for u in range(2):
        pltpu.make_async_copy(
            bufs[u], out_hbm.at[pl.ds(0, 8), pl.ds(0, W)], sems[u]).wait()
    pltpu.make_async_copy(
        buft, out_hbm.at[pl.ds(0, 8), pl.ds(TAILC0, TAILW)], semt).wait()


def _preds_body(tok_hbm, table_hbm, wb_hbm, preds_hbm,
                tok_v, emb_v, wb_v, logits_v, semg):
    wid = lax.axis_index("s") * NC + lax.axis_index("c")
    base_row = wid * BROWS

    pltpu.sync_copy(tok_hbm.at[pl.ds(base_row, BROWS)], tok_v)
    pltpu.sync_copy(wb_hbm, wb_v)

    zero16 = jnp.zeros((16,), jnp.float32)
    lane0 = lax.iota(jnp.int32, 16) == 0
    w0 = wb_v[pl.ds(0, 16)]
    w1 = wb_v[pl.ds(16, 16)]

    def prow(row, c):
        for hh in range(2):
            pltpu.async_copy(table_hbm.at[tok_v.at[row, hh]],
                             emb_v.at[pl.ds(hh * HPAD, HPAD)], semg)
        for hh in range(2):
            pltpu.make_async_copy(table_hbm.at[tok_v.at[row, hh]],
                                  emb_v.at[pl.ds(hh * HPAD, HPAD)],
                                  semg).wait()

        def acc_body(jj, acc):
            a0, a1 = acc
            return (a0 + emb_v[jj, pl.ds(0, 16)],
                    a1 + emb_v[jj, pl.ds(16, 16)])
        a0, a1 = lax.fori_loop(0, HIST_N, acc_body, (zero16, zero16))
        s = jnp.sum(a0 * w0 + a1 * w1) * (1.0 / HIST_N)
        plsc.store_scatter(logits_v, [jnp.full((16,), row, jnp.int32)],
                           jnp.full((16,), s, jnp.float32), mask=lane0)
        return c
    lax.fori_loop(0, BROWS, prow, 0)

    bvec = wb_v[pl.ds(32, 16)]
    for g in range(BROWS // 16):
        x = logits_v[pl.ds(g * 16, 16)] + bvec
        logits_v[pl.ds(g * 16, 16)] = 1.0 / (1.0 + jnp.exp(-x))
    pltpu.sync_copy(logits_v, preds_hbm.at[pl.ds(base_row, BROWS)])


@jax.jit
def kernel(token_ids, emb_table, W_dec, b):
    tok = jnp.concatenate([token_ids, token_ids[:, :8]], axis=1)
    tok = tok.astype(jnp.int32).reshape(BATCH_N, 2, HPAD)
    wb = jnp.concatenate([W_dec.reshape(EMB_N).astype(jnp.float32),
                          jnp.broadcast_to(b.astype(jnp.float32), (16,))])

    mesh = plsc.VectorSubcoreMesh(core_axis_name="c", subcore_axis_name="s")

    onehot = pl.kernel(
        _onehot_body,
        out_type=jax.ShapeDtypeStruct((BATCH_N, SLICE_V), jnp.float32),
        mesh=mesh,
        compiler_params=pltpu.CompilerParams(needs_layout_passes=False),
        scratch_types=[
            pltpu.VMEM((AROWS, 2, HPAD), jnp.int32),
            pltpu.VMEM((8, W), jnp.float32),
            pltpu.VMEM((8, W), jnp.float32),
            pltpu.VMEM((8, TAILW), jnp.float32),
            pltpu.SemaphoreType.DMA,
            pltpu.SemaphoreType.DMA,
            pltpu.SemaphoreType.DMA,
        ],
    )
    preds_k = pl.kernel(
        _preds_body,
        out_type=jax.ShapeDtypeStruct((BATCH_N,), jnp.float32),
        mesh=mesh,
        compiler_params=pltpu.CompilerParams(use_tc_tiling_on_sc=False,
                                             needs_layout_passes=False),
        scratch_types=[
            pltpu.VMEM((BROWS, 2, HPAD), jnp.int32),
            pltpu.VMEM((2 * HPAD, EMB_N), jnp.float32),
            pltpu.VMEM((48,), jnp.float32),
            pltpu.VMEM((BROWS,), jnp.float32),
            pltpu.SemaphoreType.DMA,
        ],
    )
    slices = [onehot(tok - jnp.int32(s * SLICE_V)) for s in range(NSLICE)]
    preds = preds_k(tok, emb_table, wb)
    input_vector = jnp.concatenate(slices, axis=1)
    return (input_vector, preds)


# speedup vs baseline: 3.2197x; 3.2197x over previous
import jax
import jax.numpy as jnp
from jax import lax
from jax.experimental import pallas as pl
from jax.experimental.pallas import tpu as pltpu
from jax.experimental.pallas import tpu_sc as plsc

VOCAB_N = 100000
EMB_N = 32
BATCH_N = 1024
HIST_N = 200

NC = 2
NS = 16
NW = NC * NS
AROWS = BATCH_N // NW
RGS = AROWS // 8
BROWS = BATCH_N // NW
W = 4992
NK = 20
TAILC0 = NK * W
TAILW = VOCAB_N - TAILC0
HPAD = 104
CH_OFFS = (0, 16, 32, 48, 64, 80, 88)


def _onehot_body(tok_hbm, out_hbm, tok_v, buf0, buf1, buft,
                 sem0, sem1, semt):
    wid = lax.axis_index("s") * NC + lax.axis_index("c")
    base_row = wid * AROWS
    base_rg = wid * RGS

    pltpu.sync_copy(tok_hbm.at[pl.ds(base_row, AROWS)], tok_v)

    zero16 = jnp.zeros((16,), jnp.float32)
    one16 = jnp.ones((16,), jnp.float32)

    def z_main(i, carry):
        for rr in range(8):
            buf0[rr, pl.ds(i * 16, 16)] = zero16
            buf1[rr, pl.ds(i * 16, 16)] = zero16
        return carry
    lax.fori_loop(0, W // 16, z_main, 0)

    tail_offs = tuple(range(0, TAILW - 16, 16)) + (TAILW - 16,)

    def z_tail(rr, carry):
        for off in tail_offs:
            buft[rr, pl.ds(off, 16)] = zero16
        return carry
    lax.fori_loop(0, 8, z_tail, 0)

    def scan(buf, rg, c0, cw, val16):
        def rbody(rr, carry):
            row = rg * 8 + rr
            ir = jnp.full((16,), rr, jnp.int32)
            for hh in range(2):
                for off in CH_OFFS:
                    t = tok_v[row, hh, pl.ds(off, 16)]
                    m = (t >= c0) & (t < c0 + cw)
                    ic = jnp.where(m, t - c0, 0)
                    plsc.store_scatter(buf, [ir, ic], val16, mask=m)
            return carry
        lax.fori_loop(0, 8, rbody, 0)

    bufs = (buf0, buf1)
    sems = (sem0, sem1)

    def rg_body(rg, carry):
        rgg = base_rg + rg

        def kp_body(j, c):
            for u in range(2):
                k = 2 * j + u
                c0 = k * W
                buf, sem = bufs[u], sems[u]

                @pl.when((k >= 2) | (rg > 0))
                def _reuse(buf=buf, sem=sem, k=k, c0=c0):
                    k2 = jnp.where(k >= 2, k - 2, NK - 2 + k)
                    rg2 = jnp.where(k >= 2, rg, rg - 1)
                    pltpu.make_async_copy(
                        buf, out_hbm.at[pl.ds(0, 8), pl.ds(0, W)], sem).wait()
                    scan(buf, rg2, k2 * W, W, zero16)

                scan(buf, rg, c0, W, one16)
                pltpu.async_copy(
                    buf, out_hbm.at[pl.ds(rgg * 8, 8), pl.ds(c0, W)], sem)
            return c
        lax.fori_loop(0, NK // 2, kp_body, 0)

        @pl.when(rg > 0)
        def _tail_reuse():
            pltpu.make_async_copy(
                buft, out_hbm.at[pl.ds(0, 8), pl.ds(TAILC0, TAILW)],
                semt).wait()
            scan(buft, rg - 1, TAILC0, TAILW, zero16)
        scan(buft, rg, TAILC0, TAILW, one16)
        pltpu.async_copy(
            buft, out_hbm.at[pl.ds(rgg * 8, 8), pl.ds(TAILC0, TAILW)], semt)
        return carry
    lax.fori_loop(0, RGS, rg_body, 0)

    for u in range(2):
        pltpu.make_async_copy(
            bufs[u], out_hbm.at[pl.ds(0, 8), pl.ds(0, W)], sems[u]).wait()
    pltpu.make_async_copy(
        buft, out_hbm.at[pl.ds(0, 8), pl.ds(TAILC0, TAILW)], semt).wait()


def _preds_body(tok_hbm, table_hbm, wb_hbm, preds_hbm,
                tok_v, emb_v, wb_v, logits_v, semg):
    wid = lax.axis_index("s") * NC + lax.axis_index("c")
    base_row = wid * BROWS

    pltpu.sync_copy(tok_hbm.at[pl.ds(base_row, BROWS)], tok_v)
    pltpu.sync_copy(wb_hbm, wb_v)

    zero16 = jnp.zeros((16,), jnp.float32)
    lane0 = lax.iota(jnp.int32, 16) == 0
    w0 = wb_v[pl.ds(0, 16)]
    w1 = wb_v[pl.ds(16, 16)]

    def prow(row, c):
        for hh in range(2):
            pltpu.async_copy(table_hbm.at[tok_v.at[row, hh]],
                             emb_v.at[pl.ds(hh * HPAD, HPAD)], semg)
        for hh in range(2):
            pltpu.make_async_copy(table_hbm.at[tok_v.at[row, hh]],
                                  emb_v.at[pl.ds(hh * HPAD, HPAD)],
                                  semg).wait()

        def acc_body(jj, acc):
            a0, a1 = acc
            return (a0 + emb_v[jj, pl.ds(0, 16)],
                    a1 + emb_v[jj, pl.ds(16, 16)])
        a0, a1 = lax.fori_loop(0, HIST_N, acc_body, (zero16, zero16))
        s = jnp.sum(a0 * w0 + a1 * w1) * (1.0 / HIST_N)
        plsc.store_scatter(logits_v, [jnp.full((16,), row, jnp.int32)],
                           jnp.full((16,), s, jnp.float32), mask=lane0)
        return c
    lax.fori_loop(0, BROWS, prow, 0)

    bvec = wb_v[pl.ds(32, 16)]
    for g in range(BROWS // 16):
        x = logits_v[pl.ds(g * 16, 16)] + bvec
        logits_v[pl.ds(g * 16, 16)] = 1.0 / (1.0 + jnp.exp(-x))
    pltpu.sync_copy(logits_v, preds_hbm.at[pl.ds(base_row, BROWS)])


@jax.jit
def kernel(token_ids, emb_table, W_dec, b):
    tok = jnp.concatenate([token_ids, token_ids[:, :8]], axis=1)
    tok = tok.astype(jnp.int32).reshape(BATCH_N, 2, HPAD)
    wb = jnp.concatenate([W_dec.reshape(EMB_N).astype(jnp.float32),
                          jnp.broadcast_to(b.astype(jnp.float32), (16,))])

    mesh = plsc.VectorSubcoreMesh(core_axis_name="c", subcore_axis_name="s")

    onehot = pl.kernel(
        _onehot_body,
        out_type=jax.ShapeDtypeStruct((BATCH_N, VOCAB_N), jnp.float32),
        mesh=mesh,
        compiler_params=pltpu.CompilerParams(needs_layout_passes=False),
        scratch_types=[
            pltpu.VMEM((AROWS, 2, HPAD), jnp.int32),
            pltpu.VMEM((8, W), jnp.float32),
            pltpu.VMEM((8, W), jnp.float32),
            pltpu.VMEM((8, TAILW), jnp.float32),
            pltpu.SemaphoreType.DMA,
            pltpu.SemaphoreType.DMA,
            pltpu.SemaphoreType.DMA,
        ],
    )
    preds_k = pl.kernel(
        _preds_body,
        out_type=jax.ShapeDtypeStruct((BATCH_N,), jnp.float32),
        mesh=mesh,
        compiler_params=pltpu.CompilerParams(use_tc_tiling_on_sc=False,
                                             needs_layout_passes=False),
        scratch_types=[
            pltpu.VMEM((BROWS, 2, HPAD), jnp.int32),
            pltpu.VMEM((2 * HPAD, EMB_N), jnp.float32),
            pltpu.VMEM((48,), jnp.float32),
            pltpu.VMEM((BROWS,), jnp.float32),
            pltpu.SemaphoreType.DMA,
        ],
    )
    input_vector = onehot(tok)
    preds = preds_k(tok, emb_table, wb)
    return (input_vector, preds)


# speedup vs baseline: 3.5165x; 1.0922x over previous
import jax
import jax.numpy as jnp
from jax import lax
from jax.experimental import pallas as pl
from jax.experimental.pallas import tpu as pltpu
from jax.experimental.pallas import tpu_sc as plsc

VOCAB_N = 100000
EMB_N = 32
BATCH_N = 1024
HIST_N = 200

NC = 2
NS = 16
NW = NC * NS
AROWS = BATCH_N // NW
RGS = AROWS // 8
BROWS = BATCH_N // NW
W = 6144
NK = 16
TAILC0 = NK * W
TAILW = VOCAB_N - TAILC0
HPAD = 104
HISTP = 2 * HPAD


def _onehot_body(tok_hbm, out_hbm, tok_v, buf0, buf1, buft,
                 sem0, sem1, semt):
    wid = lax.axis_index("s") * NC + lax.axis_index("c")
    base_row = wid * AROWS
    base_rg = wid * RGS

    pltpu.sync_copy(tok_hbm.at[pl.ds(base_row, AROWS)], tok_v)

    zero16 = jnp.zeros((16,), jnp.float32)
    one16 = jnp.ones((16,), jnp.float32)

    def z_main(i, carry):
        for rr in range(8):
            buf0[rr, pl.ds(i * 16, 16)] = zero16
            buf1[rr, pl.ds(i * 16, 16)] = zero16
        return carry
    lax.fori_loop(0, W // 16, z_main, 0)

    def z_tail(i, carry):
        for rr in range(8):
            buft[rr, pl.ds(i * 16, 16)] = zero16
        return carry
    lax.fori_loop(0, TAILW // 16, z_tail, 0)

    def scan(buf, rg, c0, cw, val16):
        cwu = jnp.uint32(cw)

        def rbody(rr, carry):
            row = rg * 8 + rr
            ir = jnp.full((16,), rr, jnp.int32)
            for ch in range(HISTP // 16):
                t = tok_v[row, pl.ds(ch * 16, 16)]
                ic = t - c0
                m = ic.astype(jnp.uint32) < cwu
                plsc.store_scatter(buf, [ir, ic], val16, mask=m)
            return carry
        lax.fori_loop(0, 8, rbody, 0)

    bufs = (buf0, buf1)
    sems = (sem0, sem1)

    def rg_body(rg, carry):
        rgg = base_rg + rg

        def kp_body(j, c):
            for u in range(2):
                k = 2 * j + u
                c0 = k * W
                buf, sem = bufs[u], sems[u]

                @pl.when((k >= 2) | (rg > 0))
                def _reuse(buf=buf, sem=sem, k=k, c0=c0):
                    k2 = jnp.where(k >= 2, k - 2, NK - 2 + k)
                    rg2 = jnp.where(k >= 2, rg, rg - 1)
                    pltpu.make_async_copy(
                        buf, out_hbm.at[pl.ds(0, 8), pl.ds(0, W)], sem).wait()
                    scan(buf, rg2, k2 * W, W, zero16)

                scan(buf, rg, c0, W, one16)
                pltpu.async_copy(
                    buf, out_hbm.at[pl.ds(rgg * 8, 8), pl.ds(c0, W)], sem)
            return c
        lax.fori_loop(0, NK // 2, kp_body, 0)

        @pl.when(rg > 0)
        def _tail_reuse():
            pltpu.make_async_copy(
                buft, out_hbm.at[pl.ds(0, 8), pl.ds(TAILC0, TAILW)],
                semt).wait()
            scan(buft, rg - 1, TAILC0, TAILW, zero16)
        scan(buft, rg, TAILC0, TAILW, one16)
        pltpu.async_copy(
            buft, out_hbm.at[pl.ds(rgg * 8, 8), pl.ds(TAILC0, TAILW)], semt)
        return carry
    lax.fori_loop(0, RGS, rg_body, 0)

    for u in range(2):
        pltpu.make_async_copy(
            bufs[u], out_hbm.at[pl.ds(0, 8), pl.ds(0, W)], sems[u]).wait()
    pltpu.make_async_copy(
        buft, out_hbm.at[pl.ds(0, 8), pl.ds(TAILC0, TAILW)], semt).wait()


def _preds_body(tok_hbm, table_hbm, wb_hbm, preds_hbm,
                tok_v, emb_v, wb_v, logits_v, semg):
    wid = lax.axis_index("s") * NC + lax.axis_index("c")
    base_row = wid * BROWS

    pltpu.sync_copy(tok_hbm.at[pl.ds(base_row, BROWS)], tok_v)
    pltpu.sync_copy(wb_hbm, wb_v)

    zero16 = jnp.zeros((16,), jnp.float32)
    lane0 = lax.iota(jnp.int32, 16) == 0
    w0 = wb_v[pl.ds(0, 16)]
    w1 = wb_v[pl.ds(16, 16)]

    def prow(row, c):
        for hh in range(2):
            pltpu.async_copy(table_hbm.at[tok_v.at[row, hh]],
                             emb_v.at[pl.ds(hh * HPAD, HPAD)], semg)
        for hh in range(2):
            pltpu.make_async_copy(table_hbm.at[tok_v.at[row, hh]],
                                  emb_v.at[pl.ds(hh * HPAD, HPAD)],
                                  semg).wait()

        def acc_body(jj, acc):
            a0, a1 = acc
            return (a0 + emb_v[jj, pl.ds(0, 16)],
                    a1 + emb_v[jj, pl.ds(16, 16)])
        a0, a1 = lax.fori_loop(0, HIST_N, acc_body, (zero16, zero16))
        s = jnp.sum(a0 * w0 + a1 * w1) * (1.0 / HIST_N)
        plsc.store_scatter(logits_v, [jnp.full((16,), row, jnp.int32)],
                           jnp.full((16,), s, jnp.float32), mask=lane0)
        return c
    lax.fori_loop(0, BROWS, prow, 0)

    bvec = wb_v[pl.ds(32, 16)]
    for g in range(BROWS // 16):
        x = logits_v[pl.ds(g * 16, 16)] + bvec
        logits_v[pl.ds(g * 16, 16)] = 1.0 / (1.0 + jnp.exp(-x))
    pltpu.sync_copy(logits_v, preds_hbm.at[pl.ds(base_row, BROWS)])


@jax.jit
def kernel(token_ids, emb_table, W_dec, b):
    tok208 = jnp.concatenate([token_ids, token_ids[:, :8]],
                             axis=1).astype(jnp.int32)
    tok = tok208.reshape(BATCH_N, 2, HPAD)
    wb = jnp.concatenate([W_dec.reshape(EMB_N).astype(jnp.float32),
                          jnp.broadcast_to(b.astype(jnp.float32), (16,))])

    mesh = plsc.VectorSubcoreMesh(core_axis_name="c", subcore_axis_name="s")

    onehot = pl.kernel(
        _onehot_body,
        out_type=jax.ShapeDtypeStruct((BATCH_N, VOCAB_N), jnp.float32),
        mesh=mesh,
        compiler_params=pltpu.CompilerParams(needs_layout_passes=False),
        scratch_types=[
            pltpu.VMEM((AROWS, HISTP), jnp.int32),
            pltpu.VMEM((8, W), jnp.float32),
            pltpu.VMEM((8, W), jnp.float32),
            pltpu.VMEM((8, TAILW), jnp.float32),
            pltpu.SemaphoreType.DMA,
            pltpu.SemaphoreType.DMA,
            pltpu.SemaphoreType.DMA,
        ],
    )
    preds_k = pl.kernel(
        _preds_body,
        out_type=jax.ShapeDtypeStruct((BATCH_N,), jnp.float32),
        mesh=mesh,
        compiler_params=pltpu.CompilerParams(use_tc_tiling_on_sc=False,
                                             needs_layout_passes=False),
        scratch_types=[
            pltpu.VMEM((BROWS, 2, HPAD), jnp.int32),
            pltpu.VMEM((2 * HPAD, EMB_N), jnp.float32),
            pltpu.VMEM((48,), jnp.float32),
            pltpu.VMEM((BROWS,), jnp.float32),
            pltpu.SemaphoreType.DMA,
        ],
    )
    input_vector = onehot(tok208)
    preds = preds_k(tok, emb_table, wb)
    return (input_vector, preds)
